# baseline (speedup 1.0000x reference)
import jax
import jax.numpy as jnp
import numpy as np
from jax import lax
from jax.experimental import pallas as pl
from jax.experimental.pallas import tpu as pltpu
from jax.experimental.pallas import tpu_sc as plsc

B, C, H, W = 8, 96, 224, 224
R = B * C
NC, NS = 2, 16
NW = NC * NS
M = R // NW
L = 8


def _body(x_hbm, idx_hbm, out_hbm, idx_v, sems):
    c = lax.axis_index("c")
    s = lax.axis_index("s")
    w = s * NC + c
    base = w * M
    pltpu.sync_copy(idx_hbm.at[pl.ds(base, M)], idx_v)

    lanes = lax.broadcasted_iota(jnp.int32, (16,), 0)
    vecs = [idx_v[pl.ds(8 * g, 16)] for g in range(2)]

    def src_of(j):
        vec, lane = (vecs[0], j) if j < 16 else (vecs[1], j - 8)
        return lax.reduce_max(jnp.where(lanes == lane, vec, 0), (0,))

    def copy(j):
        k = j % L
        return pltpu.make_async_copy(x_hbm.at[pl.ds(src_of(j), 1)],
                                     out_hbm.at[pl.ds(base + j, 1)],
                                     sems[k])

    for j in range(M):
        if j >= L:
            copy(j - L).wait()
        copy(j).start()
    for j in range(M - L, M):
        copy(j).wait()


@jax.jit
def kernel(x, perm):
    x3 = x.reshape(R, H, W)
    rows = jnp.arange(R, dtype=jnp.int32)
    src = (rows // C) * C + perm.astype(jnp.int32)[rows % C]

    mesh = plsc.VectorSubcoreMesh(core_axis_name="c", subcore_axis_name="s")
    out3 = pl.kernel(
        _body,
        out_type=jax.ShapeDtypeStruct((R, H, W), jnp.float32),
        mesh=mesh,
        compiler_params=pltpu.CompilerParams(use_tc_tiling_on_sc=True,
                                             needs_layout_passes=False),
        scratch_types=[
            pltpu.VMEM((M,), jnp.int32),
            [pltpu.SemaphoreType.DMA for _ in range(L)],
        ],
    )(x3, src)
    return out3.reshape(B, C, H, W)


# speedup vs baseline: 37.5088x; 37.5088x over previous
import jax
import jax.numpy as jnp
from jax import lax
from jax.experimental import pallas as pl
from jax.experimental.pallas import tpu as pltpu
from jax.experimental.pallas import tpu_sc as plsc

B, C, H, W = 8, 96, 224, 224
R = B * C
NC, NS = 2, 16
NW = NC * NS
M = R // NW


def _body(x_hbm, perm_hbm, out_hbm, idx_v, bufs, gsems, ssems):
    wid = lax.axis_index("s") * NC + lax.axis_index("c")
    base = wid * M
    batch = wid // (C // M)
    c0 = pl.multiple_of((wid % (C // M)) * M, 8)
    pltpu.sync_copy(perm_hbm.at[pl.ds(c0, M)], idx_v)

    lanes = lax.broadcasted_iota(jnp.int32, (16,), 0)
    v0 = idx_v[pl.ds(0, 16)]
    v1 = idx_v[pl.ds(8, 16)]

    def src_of(j):
        vec, lane = (v0, j) if j < 16 else (v1, j - 8)
        ch = lax.reduce_max(jnp.where(lanes == lane, vec, 0), (0,))
        return ch + batch * C

    def gather(j):
        b = j % 2
        return pltpu.async_copy(x_hbm.at[pl.ds(src_of(j), 1)], bufs[b],
                                gsems[b])

    def wait_gather(j):
        b = j % 2
        pltpu.make_async_copy(x_hbm.at[pl.ds(src_of(j), 1)], bufs[b],
                              gsems[b]).wait()

    def scatter(j):
        b = j % 2
        return pltpu.async_copy(bufs[b], out_hbm.at[pl.ds(base + j, 1)],
                                ssems[b])

    def wait_scatter(j):
        b = j % 2
        pltpu.make_async_copy(bufs[b], out_hbm.at[pl.ds(base + j, 1)],
                              ssems[b]).wait()

    gather(0)
    gather(1)
    for j in range(M):
        wait_gather(j)
        scatter(j)
        if j + 2 < M:
            wait_scatter(j)
            gather(j + 2)
    wait_scatter(M - 2)
    wait_scatter(M - 1)


@jax.jit
def kernel(x, perm):
    x3 = x.reshape(R, H, W)
    perm32 = perm.astype(jnp.int32)

    mesh = plsc.VectorSubcoreMesh(core_axis_name="c", subcore_axis_name="s")
    out3 = pl.kernel(
        _body,
        out_type=jax.ShapeDtypeStruct((R, H, W), jnp.float32),
        mesh=mesh,
        compiler_params=pltpu.CompilerParams(use_tc_tiling_on_sc=True,
                                             needs_layout_passes=False),
        scratch_types=[
            pltpu.VMEM((M,), jnp.int32),
            [pltpu.VMEM((1, H, W), jnp.float32) for _ in range(2)],
            [pltpu.SemaphoreType.DMA for _ in range(2)],
            [pltpu.SemaphoreType.DMA for _ in range(2)],
        ],
    )(x3, perm32)
    return out3.reshape(B, C, H, W)


# speedup vs baseline: 37.5788x; 1.0019x over previous
import jax
import jax.numpy as jnp
from jax import lax
from jax.experimental import pallas as pl
from jax.experimental.pallas import tpu as pltpu
from jax.experimental.pallas import tpu_sc as plsc

B, C, H, W = 8, 96, 224, 224
R = B * C
NC, NS = 2, 16
NW = NC * NS
M = R // NW


def _body(x_hbm, perm_hbm, out_hbm, idx_v, bufs, gsems, ssems):
    wid = lax.axis_index("s") * NC + lax.axis_index("c")
    base = wid * M
    batch = wid // (C // M)
    c0 = pl.multiple_of((wid % (C // M)) * M, 8)
    pltpu.sync_copy(perm_hbm.at[pl.ds(c0, M)], idx_v)

    lanes = lax.broadcasted_iota(jnp.int32, (16,), 0)
    v0 = idx_v[pl.ds(0, 16)]
    v1 = idx_v[pl.ds(8, 16)]

    def src_of(j):
        vec, lane = (v0, j) if j < 16 else (v1, j - 8)
        ch = lax.reduce_max(jnp.where(lanes == lane, vec, 0), (0,))
        return ch + batch * C

    srcs = [src_of(j) for j in range(M)]

    def gather(j):
        b = j % 2
        return pltpu.async_copy(x_hbm.at[pl.ds(srcs[j], 1)], bufs[b],
                                gsems[b])

    def wait_gather(j):
        b = j % 2
        pltpu.make_async_copy(x_hbm.at[pl.ds(srcs[j], 1)], bufs[b],
                              gsems[b]).wait()

    def scatter(j):
        b = j % 2
        return pltpu.async_copy(bufs[b], out_hbm.at[pl.ds(base + j, 1)],
                                ssems[b])

    def wait_scatter(j):
        b = j % 2
        pltpu.make_async_copy(bufs[b], out_hbm.at[pl.ds(base + j, 1)],
                              ssems[b]).wait()

    gather(0)
    gather(1)
    for j in range(M):
        wait_gather(j)
        scatter(j)
        if j + 2 < M:
            wait_scatter(j)
            gather(j + 2)
    wait_scatter(M - 2)
    wait_scatter(M - 1)


@jax.jit
def kernel(x, perm):
    x3 = x.reshape(R, H, W)
    perm32 = perm.astype(jnp.int32)

    mesh = plsc.VectorSubcoreMesh(core_axis_name="c", subcore_axis_name="s")
    out3 = pl.kernel(
        _body,
        out_type=jax.ShapeDtypeStruct((R, H, W), jnp.float32),
        mesh=mesh,
        compiler_params=pltpu.CompilerParams(use_tc_tiling_on_sc=True,
                                             needs_layout_passes=False),
        scratch_types=[
            pltpu.VMEM((M,), jnp.int32),
            [pltpu.VMEM((1, H, W), jnp.float32) for _ in range(2)],
            [pltpu.SemaphoreType.DMA for _ in range(2)],
            [pltpu.SemaphoreType.DMA for _ in range(2)],
        ],
    )(x3, perm32)
    return out3.reshape(B, C, H, W)


# speedup vs baseline: 37.6199x; 1.0011x over previous
import jax
import jax.numpy as jnp
from jax import lax
from jax.experimental import pallas as pl
from jax.experimental.pallas import tpu as pltpu
from jax.experimental.pallas import tpu_sc as plsc

B, C, H, W = 8, 96, 224, 224
R = B * C
NC, NS = 2, 16
NW = NC * NS
M = R // NW


def _body(x_hbm, perm_hbm, out_hbm, idx_v, bufs, gsems, ssems):
    wid = lax.axis_index("c") * NS + lax.axis_index("s")
    base = wid * M
    batch = wid // (C // M)
    c0 = pl.multiple_of((wid % (C // M)) * M, 8)
    pltpu.sync_copy(perm_hbm.at[pl.ds(c0, M)], idx_v)

    lanes = lax.broadcasted_iota(jnp.int32, (16,), 0)
    v0 = idx_v[pl.ds(0, 16)]
    v1 = idx_v[pl.ds(8, 16)]

    def src_of(j):
        vec, lane = (v0, j) if j < 16 else (v1, j - 8)
        ch = lax.reduce_max(jnp.where(lanes == lane, vec, 0), (0,))
        return ch + batch * C

    srcs = [src_of(j) for j in range(M)]

    def gather(j):
        b = j % 2
        return pltpu.async_copy(x_hbm.at[pl.ds(srcs[j], 1)], bufs[b],
                                gsems[b])

    def wait_gather(j):
        b = j % 2
        pltpu.make_async_copy(x_hbm.at[pl.ds(srcs[j], 1)], bufs[b],
                              gsems[b]).wait()

    def scatter(j):
        b = j % 2
        return pltpu.async_copy(bufs[b], out_hbm.at[pl.ds(base + j, 1)],
                                ssems[b])

    def wait_scatter(j):
        b = j % 2
        pltpu.make_async_copy(bufs[b], out_hbm.at[pl.ds(base + j, 1)],
                              ssems[b]).wait()

    gather(0)
    gather(1)
    for j in range(M):
        wait_gather(j)
        scatter(j)
        if j + 2 < M:
            wait_scatter(j)
            gather(j + 2)
    wait_scatter(M - 2)
    wait_scatter(M - 1)


@jax.jit
def kernel(x, perm):
    x3 = x.reshape(R, H, W)
    perm32 = perm.astype(jnp.int32)

    mesh = plsc.VectorSubcoreMesh(core_axis_name="c", subcore_axis_name="s")
    out3 = pl.kernel(
        _body,
        out_type=jax.ShapeDtypeStruct((R, H, W), jnp.float32),
        mesh=mesh,
        compiler_params=pltpu.CompilerParams(use_tc_tiling_on_sc=True,
                                             needs_layout_passes=False),
        scratch_types=[
            pltpu.VMEM((M,), jnp.int32),
            [pltpu.VMEM((1, H, W), jnp.float32) for _ in range(2)],
            [pltpu.SemaphoreType.DMA for _ in range(2)],
            [pltpu.SemaphoreType.DMA for _ in range(2)],
        ],
    )(x3, perm32)
    return out3.reshape(B, C, H, W)
